# Initial kernel scaffold; baseline (speedup 1.0000x reference)
import jax
import jax.numpy as jnp
from jax.experimental import pallas as pl


def kernel(x, edge_index, W1_pool, b1_pool, W1_neigh, W1_self, b1, W2_pool, b2_pool, W2_neigh, W2_self, b2, W3_neigh, W3_self, b3):
    raise NotImplementedError("write your pallas kernel here")


# speedup vs baseline: 3.1617x; 3.1617x over previous
import functools

import jax
import jax.numpy as jnp
from jax import lax
from jax.experimental import pallas as pl
from jax.experimental.pallas import tpu as pltpu
from jax.experimental.pallas import tpu_sc as plsc

_N = 50000
_E = 800000
_NC = 2
_NS = 16
_NW = _NC * _NS
_R = 1568
_NPAD = _NW * _R
_CHUNK = 1600
_NCHUNK = _E // _CHUNK
_G = 128
_BUF = 1792


_EL = _E + _G


def _make_segmax_scan():
  nfeat = 64
  nfv = nfeat // 16
  mesh = plsc.VectorSubcoreMesh(core_axis_name="c", subcore_axis_name="s")
  out_type = (jax.ShapeDtypeStruct((_NPAD * nfeat,), jnp.float32),
              jax.ShapeDtypeStruct((_NPAD,), jnp.float32),
              jax.ShapeDtypeStruct((_NW, _EL), jnp.int32),
              jax.ShapeDtypeStruct((_NW, _EL), jnp.int32),
              jax.ShapeDtypeStruct((_NW, 16), jnp.int32))
  scratch = (
      pltpu.VMEM(((_R + 1) * nfeat,), jnp.float32),
      pltpu.VMEM((_CHUNK,), jnp.int32),
      pltpu.VMEM((_CHUNK,), jnp.int32),
      pltpu.VMEM((_BUF,), jnp.int32),
      pltpu.VMEM((_BUF,), jnp.int32),
      pltpu.VMEM((_G, nfeat), jnp.float32),
      pltpu.VMEM((16,), jnp.int32),
      pltpu.SemaphoreType.DMA,
      pltpu.SemaphoreType.DMA,
      pltpu.VMEM((_R + 16,), jnp.float32),
  )

  def body(p_hbm, src_hbm, dst_hbm, agg_hbm, deg_hbm, els_hbm, eld_hbm,
           cnt_hbm, acc, sbuf, dbuf, selsrc, seldst, rows, cstage, gsem,
           wsem, degacc):
    wid = lax.axis_index("s") * _NC + lax.axis_index("c")
    lo = wid * _R
    zf = jnp.zeros((16,), jnp.float32)
    zi = jnp.zeros((16,), jnp.int32)
    di = jnp.full((16,), _R, jnp.int32)
    ones = jnp.ones((16,), jnp.float32)
    iot = lax.iota(jnp.int32, 16)

    def zero_acc(i, _):
      acc[pl.ds(pl.multiple_of(i * 16, 8), 16)] = zf
      return 0

    lax.fori_loop(0, (_R + 1) * nfv, zero_acc, 0)

    def zero_deg(i, _):
      degacc[pl.ds(pl.multiple_of(i * 16, 8), 16)] = zf
      return 0

    lax.fori_loop(0, (_R + 16) // 16, zero_deg, 0)

    def drain_writes(n2):
      def dw(i, _):
        pltpu.make_async_copy(selsrc.at[pl.ds(0, _G)],
                              els_hbm.at[wid, pl.ds(0, _G)], wsem).wait()
        return 0
      lax.fori_loop(0, n2, dw, 0)

    def do_batch(ro, wcnt):
      ro = pl.multiple_of(ro, 8)
      idx = selsrc.at[pl.ds(ro, _G)]
      wo = pl.multiple_of(wcnt, 8)
      pltpu.async_copy(idx, els_hbm.at[wid, pl.ds(wo, _G)], wsem)
      pltpu.async_copy(seldst.at[pl.ds(ro, _G)], eld_hbm.at[wid, pl.ds(wo, _G)],
                       wsem)
      pltpu.async_copy(p_hbm.at[idx], rows, gsem).wait()

      def rmw16(g, _):
        dvec = seldst[pl.ds(pl.multiple_of(ro + g * 16, 8), 16)]
        plsc.addupdate_scatter(degacc, [dvec], ones)
        for l in range(16):
          b = dvec[l] * nfeat
          j = g * 16 + l
          for c in range(nfv):
            sl = pl.ds(pl.multiple_of(b + c * 16, 8), 16)
            acc[sl] = jnp.maximum(acc[sl], rows[j, pl.ds(c * 16, 16)])
        return 0

      lax.fori_loop(0, _G // 16, rmw16, 0)

    def scan(i, pos):
      base = pl.multiple_of(i * 16, 8)
      vd = dbuf[pl.ds(base, 16)]
      vs = sbuf[pl.ds(base, 16)]
      m = (vd >= lo) & (vd < lo + _R)
      plsc.store_compressed(selsrc.at[pl.ds(pos, 16)], vs, mask=m)
      plsc.store_compressed(seldst.at[pl.ds(pos, 16)], vd - lo, mask=m)
      cnt = plsc.all_reduce_population_count(m)[0]
      return pos + cnt

    def chunk(ci, state):
      pos, wcnt = state
      off = pl.multiple_of(ci * _CHUNK, 8)
      pltpu.sync_copy(src_hbm.at[pl.ds(off, _CHUNK)], sbuf)
      pltpu.sync_copy(dst_hbm.at[pl.ds(off, _CHUNK)], dbuf)
      pos = lax.fori_loop(0, _CHUNK // 16, scan, pos)

      def dcond(c):
        p, ro, w = c
        return p - ro >= _G

      def dbody(c):
        p, ro, w = c
        do_batch(ro, w)
        return (p, ro + _G, w + _G)

      pos, ro, wcnt = lax.while_loop(dcond, dbody, (pos, jnp.int32(0), wcnt))
      drain_writes((ro // _G) * 2)

      @pl.when(ro > 0)
      def _():
        for k in range(8):
          src_k = pl.ds(pl.multiple_of(ro + k * 16, 8), 16)
          dst_k = pl.ds(k * 16, 16)
          selsrc[dst_k] = selsrc[src_k]
          seldst[dst_k] = seldst[src_k]

      return pos - ro, wcnt

    pos, wcnt = lax.fori_loop(0, _NCHUNK, chunk,
                              (jnp.int32(0), jnp.int32(0)))

    @pl.when(pos > 0)
    def _():
      for k in range(8):
        plsc.store_scatter(selsrc, [pos + iot + k * 16], zi)
        plsc.store_scatter(seldst, [pos + iot + k * 16], di)
      do_batch(0, wcnt)
      drain_writes(2)

    wtot = jnp.where(pos > 0, wcnt + _G, wcnt)
    cstage[pl.ds(0, 16)] = jnp.broadcast_to(wtot, (16,))
    pltpu.sync_copy(cstage, cnt_hbm.at[wid])
    pltpu.sync_copy(acc.at[pl.ds(0, _R * nfeat)],
                    agg_hbm.at[pl.ds(pl.multiple_of(lo * nfeat, 8),
                                     _R * nfeat)])
    pltpu.sync_copy(degacc.at[pl.ds(0, _R)],
                    deg_hbm.at[pl.ds(pl.multiple_of(lo, 8), _R)])

  return pl.kernel(body, out_type=out_type, mesh=mesh,
                   scratch_types=scratch,
                   compiler_params=pltpu.CompilerParams(
                       use_tc_tiling_on_sc=False, needs_layout_passes=False))


def _make_segmax_list():
  nfeat = 64
  nfv = nfeat // 16
  mesh = plsc.VectorSubcoreMesh(core_axis_name="c", subcore_axis_name="s")
  out_type = jax.ShapeDtypeStruct((_NPAD * nfeat,), jnp.float32)
  scratch = (
      pltpu.VMEM(((_R + 1) * nfeat,), jnp.float32),
      pltpu.VMEM((2, _G), jnp.int32),
      pltpu.VMEM((2, _G), jnp.int32),
      pltpu.VMEM((2, _G, nfeat), jnp.float32),
      pltpu.VMEM((16,), jnp.int32),
      pltpu.SemaphoreType.DMA,
      pltpu.SemaphoreType.DMA,
  )

  def body(p_hbm, els_hbm, eld_hbm, cnt_hbm, agg_hbm, acc, idxb, dlocb,
           rows, cstage, sem0, sem1):
    wid = lax.axis_index("s") * _NC + lax.axis_index("c")
    lo = wid * _R
    zf = jnp.zeros((16,), jnp.float32)
    sems = (sem0, sem1)

    def zero_acc(i, _):
      acc[pl.ds(pl.multiple_of(i * 16, 8), 16)] = zf
      return 0

    lax.fori_loop(0, (_R + 1) * nfv, zero_acc, 0)

    pltpu.sync_copy(cnt_hbm.at[wid], cstage)
    cvec = cstage[pl.ds(0, 16)]
    nb = cvec[0] // _G

    def fire(b, par):
      off = pl.multiple_of(b * _G, 8)
      pltpu.sync_copy(els_hbm.at[wid, pl.ds(off, _G)], idxb.at[par])
      pltpu.sync_copy(eld_hbm.at[wid, pl.ds(off, _G)], dlocb.at[par])
      pltpu.async_copy(p_hbm.at[idxb.at[par]], rows.at[par], sems[par])

    @pl.when(nb > 0)
    def _():
      fire(jnp.int32(0), 0)

    def batch(b, _):
      for par in (0, 1):
        @pl.when((b & 1) == par)
        def _():
          @pl.when(b + 1 < nb)
          def _():
            fire(b + 1, 1 - par)
          pltpu.make_async_copy(p_hbm.at[idxb.at[par]], rows.at[par],
                                sems[par]).wait()

          def rmw16(g, _):
            dvec = dlocb[par, pl.ds(pl.multiple_of(g * 16, 8), 16)]
            for l in range(16):
              base = dvec[l] * nfeat
              j = g * 16 + l
              for c in range(nfv):
                sl = pl.ds(pl.multiple_of(base + c * 16, 8), 16)
                acc[sl] = jnp.maximum(acc[sl], rows[par, j, pl.ds(c * 16, 16)])
            return 0

          lax.fori_loop(0, _G // 16, rmw16, 0)
      return 0

    lax.fori_loop(0, nb, batch, 0)

    pltpu.sync_copy(acc.at[pl.ds(0, _R * nfeat)],
                    agg_hbm.at[pl.ds(pl.multiple_of(lo * nfeat, 8),
                                     _R * nfeat)])

  return pl.kernel(body, out_type=out_type, mesh=mesh,
                   scratch_types=scratch,
                   compiler_params=pltpu.CompilerParams(
                       use_tc_tiling_on_sc=False, needs_layout_passes=False))


_NB3 = _E // _G
_NB3_BASE = _NB3 // _NS
_NB3_REM = _NB3 - _NB3_BASE * _NS
_H = 25088
_HS = _H // _NS


def _make_segsum():
  mesh = plsc.VectorSubcoreMesh(core_axis_name="c", subcore_axis_name="s")
  out_type = jax.ShapeDtypeStruct((_NC, _H, 32), jnp.float32)
  scratch = (
      pltpu.VMEM((_HS, 32), jnp.float32),
      pltpu.VMEM((3, 2 * _G), jnp.int32),
      pltpu.VMEM((2, _G), jnp.int32),
      pltpu.VMEM((2, _G, 32), jnp.float32),
      pltpu.VMEM_SHARED((_H + 8, 32), jnp.float32),
      pltpu.SemaphoreType.DMA,
      pltpu.SemaphoreType.DMA,
      pltpu.SemaphoreType.DMA,
      pltpu.SemaphoreType.DMA,
      pltpu.SemaphoreType.DMA,
      pltpu.SemaphoreType.DMA,
  )

  def body(h_hbm, e2_hbm, out_hbm, zbuf, ebuf, dlocb, rows, shacc,
           semi0, semi1, semi2, semg0, semg1, sems):
    cid = lax.axis_index("c")
    sid = lax.axis_index("s")
    zf = jnp.zeros((16,), jnp.float32)
    clo = cid * _H
    semi = (semi0, semi1, semi2)
    semg = (semg0, semg1)

    def zero_zbuf(i, _):
      zbuf[i, pl.ds(0, 16)] = zf
      zbuf[i, pl.ds(16, 16)] = zf
      return 0

    lax.fori_loop(0, _HS, zero_zbuf, 0)
    pltpu.sync_copy(zbuf, shacc.at[pl.ds(sid * _HS, _HS)])

    @pl.when(sid == 0)
    def _():
      pltpu.sync_copy(zbuf.at[pl.ds(0, 8)], shacc.at[pl.ds(_H, 8)])

    plsc.subcore_barrier()

    my_nb = _NB3_BASE + (sid < _NB3_REM).astype(jnp.int32)
    my_start = sid * _NB3_BASE + jnp.minimum(sid, _NB3_REM)

    def idxfire(b, sl):
      pltpu.async_copy(e2_hbm.at[my_start + b], ebuf.at[sl], semi[sl])

    def scat_drain(par):
      pltpu.make_async_copy(rows.at[par], shacc.at[dlocb.at[par]],
                            sems).wait()

    def gfire(b, sl, par):
      pltpu.make_async_copy(e2_hbm.at[my_start + b], ebuf.at[sl],
                            semi[sl]).wait()

      @pl.when(b >= 2)
      def _():
        scat_drain(par)

      for k in range(_G // 16):
        d = ebuf[sl, pl.ds(_G + k * 16, 16)] - clo
        m = (d >= 0) & (d < _H)
        dlocb[par, pl.ds(k * 16, 16)] = jnp.where(m, d, _H)
      pltpu.async_copy(h_hbm.at[ebuf.at[sl, pl.ds(0, _G)]], rows.at[par],
                       semg[par])

    def process(par):
      pltpu.make_async_copy(h_hbm.at[ebuf.at[0, pl.ds(0, _G)]],
                            rows.at[par], semg[par]).wait()
      pltpu.async_copy(rows.at[par], shacc.at[dlocb.at[par]], sems,
                       add=True)

    @pl.when(my_nb > 0)
    def _():
      idxfire(jnp.int32(0), 0)

    @pl.when(my_nb > 1)
    def _():
      idxfire(jnp.int32(1), 1)

    def batch(i, _):
      for sl in (0, 1, 2):
        @pl.when(lax.rem(i, 3) == sl)
        def _():
          for par in (0, 1):
            @pl.when((i & 1) == par)
            def _():
              gfire(i, sl, par)

              @pl.when(i + 2 < my_nb)
              def _():
                idxfire(i + 2, (sl + 2) % 3)

              process(par)
      return 0

    lax.fori_loop(0, my_nb, batch, 0)

    @pl.when(my_nb > 1)
    def _():
      scat_drain(0)

    @pl.when(my_nb > 0)
    def _():
      scat_drain(1)

    plsc.subcore_barrier()
    pltpu.sync_copy(shacc.at[pl.ds(sid * _HS, _HS)],
                    out_hbm.at[cid, pl.ds(sid * _HS, _HS)])

  return pl.kernel(body, out_type=out_type, mesh=mesh,
                   scratch_types=scratch,
                   compiler_params=pltpu.CompilerParams(
                       use_tc_tiling_on_sc=False, needs_layout_passes=False))


_segmax_scan = _make_segmax_scan()
_segmax_list = _make_segmax_list()
_segsum = _make_segsum()

_BLK = 2000
_GRID = _N // _BLK


def _row_spec(cols):
  return pl.BlockSpec((_BLK, cols), lambda i: (i, 0))


def _full_spec(r, c):
  return pl.BlockSpec((r, c), lambda i: (0, 0))


def _mm_relu_body(x_ref, w_ref, b_ref, o_ref):
  o_ref[...] = jnp.maximum(
      jnp.dot(x_ref[...], w_ref[...], preferred_element_type=jnp.float32, precision=lax.Precision.HIGHEST)
      + b_ref[...], 0.0)


_mm_relu = pl.pallas_call(
    _mm_relu_body,
    grid=(_GRID,),
    in_specs=[_row_spec(64), _full_spec(64, 64), _full_spec(1, 64)],
    out_specs=_row_spec(64),
    out_shape=jax.ShapeDtypeStruct((_N, 64), jnp.float32),
)


def _layer1_body(x_ref, a_ref, ws_ref, wn_ref, b_ref, wp_ref, bp_ref,
                 h_ref, p_ref):
  h = (jnp.dot(x_ref[...], ws_ref[...], preferred_element_type=jnp.float32, precision=lax.Precision.HIGHEST)
       + jnp.dot(a_ref[...], wn_ref[...], preferred_element_type=jnp.float32, precision=lax.Precision.HIGHEST)
       + b_ref[...])
  h = jnp.maximum(h, 0.0)
  h_ref[...] = h
  p_ref[...] = jnp.maximum(
      jnp.dot(h, wp_ref[...], preferred_element_type=jnp.float32, precision=lax.Precision.HIGHEST)
      + bp_ref[...], 0.0)


_layer1 = pl.pallas_call(
    _layer1_body,
    grid=(_GRID,),
    in_specs=[_row_spec(64), _row_spec(64), _full_spec(64, 64),
              _full_spec(64, 64), _full_spec(1, 64), _full_spec(64, 64),
              _full_spec(1, 64)],
    out_specs=[_row_spec(64), _row_spec(64)],
    out_shape=[jax.ShapeDtypeStruct((_N, 64), jnp.float32),
               jax.ShapeDtypeStruct((_N, 64), jnp.float32)],
)


def _layer2_body(h_ref, a_ref, ws_ref, wn_ref, b_ref, o_ref):
  o_ref[...] = (
      jnp.dot(h_ref[...], ws_ref[...], preferred_element_type=jnp.float32, precision=lax.Precision.HIGHEST)
      + jnp.dot(a_ref[...], wn_ref[...], preferred_element_type=jnp.float32, precision=lax.Precision.HIGHEST)
      + b_ref[...])


_layer2 = pl.pallas_call(
    _layer2_body,
    grid=(_GRID,),
    in_specs=[_row_spec(64), _row_spec(64), _full_spec(64, 32),
              _full_spec(64, 32), _full_spec(1, 32)],
    out_specs=_row_spec(32),
    out_shape=jax.ShapeDtypeStruct((_N, 32), jnp.float32),
)


def _layer3_body(h_ref, s_ref, d_ref, ws_ref, wn_ref, b_ref, o_ref):
  agg = s_ref[...] / jnp.maximum(d_ref[...], 1.0)
  o_ref[...] = (
      jnp.dot(h_ref[...], ws_ref[...], preferred_element_type=jnp.float32, precision=lax.Precision.HIGHEST)
      + jnp.dot(agg, wn_ref[...], preferred_element_type=jnp.float32, precision=lax.Precision.HIGHEST)
      + b_ref[...])


_layer3 = pl.pallas_call(
    _layer3_body,
    grid=(_GRID,),
    in_specs=[_row_spec(32), _row_spec(32),
              pl.BlockSpec((_BLK, 1), lambda i: (i, 0)), _full_spec(32, 32),
              _full_spec(32, 32), _full_spec(1, 32)],
    out_specs=_row_spec(32),
    out_shape=jax.ShapeDtypeStruct((_N, 32), jnp.float32),
)


@jax.jit
def _impl(x, edge_index, W1_pool, b1_pool, W1_neigh, W1_self, b1,
          W2_pool, b2_pool, W2_neigh, W2_self, b2, W3_neigh, W3_self, b3):
  src = edge_index[0]
  dst = edge_index[1]
  p1 = _mm_relu(x, W1_pool, b1_pool.reshape(1, 64))
  agg1f, deg, els, eld, cnts = _segmax_scan(p1, src, dst)
  agg1 = agg1f.reshape(_NPAD, 64)[:_N]
  h1, p2 = _layer1(x, agg1, W1_self, W1_neigh, b1.reshape(1, 64),
                   W2_pool, b2_pool.reshape(1, 64))
  agg2f = _segmax_list(p2, els, eld, cnts)
  agg2 = agg2f.reshape(_NPAD, 64)[:_N]
  h2 = _layer2(h1, agg2, W2_self, W2_neigh, b2.reshape(1, 32))
  e2 = jnp.concatenate([src.reshape(_NB3, _G), dst.reshape(_NB3, _G)],
                       axis=1)
  sums = _segsum(h2, e2)
  s3 = jnp.concatenate([sums[0], sums[1]], axis=0)[:_N]
  return _layer3(h2, s3, deg[:_N].reshape(_N, 1),
                 W3_self, W3_neigh, b3.reshape(1, 32))


def kernel(x, edge_index, W1_pool, b1_pool, W1_neigh, W1_self, b1,
           W2_pool, b2_pool, W2_neigh, W2_self, b2, W3_neigh, W3_self, b3):
  return _impl(x, edge_index, W1_pool, b1_pool, W1_neigh, W1_self, b1,
               W2_pool, b2_pool, W2_neigh, W2_self, b2,
               W3_neigh, W3_self, b3)
